# baseline (speedup 1.0000x reference)
import functools

import jax
import jax.numpy as jnp
from jax import lax
from jax.experimental import pallas as pl
from jax.experimental.pallas import tpu as pltpu
from jax.experimental.pallas import tpu_sc as plsc

_NULL_INDEX = 100000

_NUM_CORES = 2
_NUM_SUBCORES = 16
_LANES = 16
_NW = _NUM_CORES * _NUM_SUBCORES

_IDX_CHUNK = 128


def _build_embed(B, D):
  assert B % (8 * _NW) == 0
  bpw = B // _NW
  nchunk = bpw // _IDX_CHUNK
  assert nchunk * _IDX_CHUNK == bpw
  ncg = D // _LANES

  mesh = plsc.VectorSubcoreMesh(core_axis_name="c", subcore_axis_name="s")

  @functools.partial(
      pl.kernel,
      mesh=mesh,
      out_type=jax.ShapeDtypeStruct((B, D), jnp.float32),
      compiler_params=pltpu.CompilerParams(use_tc_tiling_on_sc=False,
                                           needs_layout_passes=False),
      scratch_types=[
          pltpu.VMEM((bpw,), jnp.int8),
          pltpu.VMEM((bpw // 4,), jnp.int32),
          pltpu.VMEM((nchunk, _IDX_CHUNK), jnp.int32),
          pltpu.VMEM((1, D), jnp.float32),
          pltpu.VMEM((bpw, D), jnp.float32),
          pltpu.SemaphoreType.DMA,
      ],
  )
  def embed(labels_hbm, drop_hbm, table_hbm, out_hbm,
            drop8_v, dropw_v, idx_v, null_v, rows_v, sem):
    wid = lax.axis_index("s") * _NUM_CORES + lax.axis_index("c")
    base = wid * bpw
    for j in range(nchunk):
      pltpu.sync_copy(labels_hbm.at[pl.ds(base + j * _IDX_CHUNK, _IDX_CHUNK)],
                      idx_v.at[j])
    pltpu.sync_copy(drop_hbm.at[pl.ds(base, bpw)], drop8_v)
    pltpu.sync_copy(table_hbm.at[pl.ds(_NULL_INDEX, 1)], null_v)
    copies = []
    for j in range(nchunk):
      copies.append(
          pltpu.async_copy(
              table_hbm.at[idx_v.at[j]],
              rows_v.at[pl.ds(j * _IDX_CHUNK, _IDX_CHUNK)],
              sem,
          ))
    for g in range(bpw // 64):
      w16 = plsc.bitcast(drop8_v[pl.ds(g * 64, 64)], jnp.int32)
      dropw_v[pl.ds(g * _LANES, _LANES)] = w16
    for c in copies:
      c.wait()

    null_cg = [null_v[0, pl.ds(cg * _LANES, _LANES)] for cg in range(ncg)]

    def blend_group(i, carry):
      for j in range(_LANES):
        word_idx = jnp.full((_LANES,), i * 4 + j // 4, jnp.int32)
        flagw = plsc.load_gather(dropw_v, [word_idx])
        flag = (flagw >> (8 * (j % 4))) & 0xFF
        pred = flag != 0
        r = i * _LANES + j
        for cg in range(ncg):
          cur = rows_v[r, pl.ds(cg * _LANES, _LANES)]
          rows_v[r, pl.ds(cg * _LANES, _LANES)] = jnp.where(
              pred, null_cg[cg], cur)
      return carry

    lax.fori_loop(0, bpw // _LANES, blend_group, 0)
    pltpu.sync_copy(rows_v, out_hbm.at[pl.ds(base, bpw)])

  return embed


@jax.jit
def _embed_call(labels, drop8, table):
  B, = labels.shape
  _, D = table.shape
  return _build_embed(B, D)(labels, drop8, table)


def kernel(labels, drop_labels, embedding_table):
  labels = labels.astype(jnp.int32)
  drop8 = drop_labels.view(jnp.int8)
  return _embed_call(labels, drop8, embedding_table)


# speedup vs baseline: 1.1274x; 1.1274x over previous
import functools

import jax
import jax.numpy as jnp
from jax import lax
from jax.experimental import pallas as pl
from jax.experimental.pallas import tpu as pltpu
from jax.experimental.pallas import tpu_sc as plsc

_NULL_INDEX = 100000

_NUM_CORES = 2
_NUM_SUBCORES = 16
_LANES = 16
_NW = _NUM_CORES * _NUM_SUBCORES

_IDX_CHUNK = 128
_DPAD = 128


def _build_embed(B, D):
  assert B % (8 * _NW) == 0
  bpw = B // _NW
  nchunk = bpw // _IDX_CHUNK
  assert nchunk * _IDX_CHUNK == bpw
  ncg = D // _LANES

  mesh = plsc.VectorSubcoreMesh(core_axis_name="c", subcore_axis_name="s")

  @functools.partial(
      pl.kernel,
      mesh=mesh,
      out_type=jax.ShapeDtypeStruct((B, _DPAD), jnp.float32),
      compiler_params=pltpu.CompilerParams(use_tc_tiling_on_sc=True,
                                           needs_layout_passes=False),
      scratch_types=[
          pltpu.VMEM((bpw // 4,), jnp.int32),
          pltpu.VMEM((nchunk, _IDX_CHUNK), jnp.int32),
          pltpu.VMEM((1, _DPAD), jnp.float32),
          pltpu.VMEM((bpw, _DPAD), jnp.float32),
          pltpu.SemaphoreType.DMA,
      ],
  )
  def embed(labels_hbm, dropw_hbm, table_hbm, out_hbm,
            dropw_v, idx_v, null_v, rows_v, sem):
    wid = lax.axis_index("s") * _NUM_CORES + lax.axis_index("c")
    base = pl.multiple_of(wid * bpw, bpw)
    base4 = pl.multiple_of(wid * (bpw // 4), bpw // 4)
    for j in range(nchunk):
      pltpu.sync_copy(labels_hbm.at[pl.ds(base + j * _IDX_CHUNK, _IDX_CHUNK)],
                      idx_v.at[j])
    pltpu.sync_copy(dropw_hbm.at[pl.ds(base4, bpw // 4)], dropw_v)
    pltpu.sync_copy(table_hbm.at[pl.ds(_NULL_INDEX, 1)], null_v)
    copies = []
    for j in range(nchunk):
      copies.append(
          pltpu.async_copy(
              table_hbm.at[idx_v.at[j]],
              rows_v.at[pl.ds(j * _IDX_CHUNK, _IDX_CHUNK)],
              sem,
          ))
    for c in copies:
      c.wait()

    null_cg = [null_v[0, pl.ds(cg * _LANES, _LANES)] for cg in range(ncg)]

    def blend_group(i, carry):
      for j in range(_LANES):
        word_idx = jnp.full((_LANES,), i * 4 + j // 4, jnp.int32)
        flagw = plsc.load_gather(dropw_v, [word_idx])
        flag = (flagw >> (8 * (j % 4))) & 0xFF
        pred = flag != 0
        r = i * _LANES + j
        for cg in range(ncg):
          cur = rows_v[r, pl.ds(cg * _LANES, _LANES)]
          rows_v[r, pl.ds(cg * _LANES, _LANES)] = jnp.where(
              pred, null_cg[cg], cur)
      return carry

    lax.fori_loop(0, bpw // _LANES, blend_group, 0)
    pltpu.sync_copy(rows_v, out_hbm.at[pl.ds(base, bpw)])

  return embed


def _embed_call(labels, dropw, table_pad):
  B, = labels.shape
  out_pad = _build_embed(B, 64)(labels, dropw, table_pad)
  return out_pad[:, :64]


def kernel(labels, drop_labels, embedding_table):
  labels = labels.astype(jnp.int32)
  drop8 = drop_labels.view(jnp.int8)
  dropw = jax.lax.bitcast_convert_type(
      drop8.reshape(drop8.shape[0] // 4, 4), jnp.int32)
  table_pad = jnp.pad(embedding_table, ((0, 0), (0, _DPAD - 64)))
  return _embed_call(labels, dropw, table_pad)


# speedup vs baseline: 1.1296x; 1.0020x over previous
import functools

import jax
import jax.numpy as jnp
from jax import lax
from jax.experimental import pallas as pl
from jax.experimental.pallas import tpu as pltpu
from jax.experimental.pallas import tpu_sc as plsc

_NULL_INDEX = 100000

_NUM_CORES = 2
_NUM_SUBCORES = 16
_LANES = 16
_NW = _NUM_CORES * _NUM_SUBCORES

_IDX_CHUNK = 128
_DPAD = 128


def _build_embed(B, D):
  assert B % (8 * _NW) == 0
  bpw = B // _NW
  nchunk = bpw // _IDX_CHUNK
  assert nchunk * _IDX_CHUNK == bpw
  ncg = D // _LANES

  mesh = plsc.VectorSubcoreMesh(core_axis_name="c", subcore_axis_name="s")

  @functools.partial(
      pl.kernel,
      mesh=mesh,
      out_type=jax.ShapeDtypeStruct((B, _DPAD), jnp.float32),
      compiler_params=pltpu.CompilerParams(use_tc_tiling_on_sc=True,
                                           needs_layout_passes=False),
      scratch_types=[
          pltpu.VMEM((bpw // 4,), jnp.int32),
          pltpu.VMEM((nchunk, _IDX_CHUNK), jnp.int32),
          pltpu.VMEM((1, _DPAD), jnp.float32),
          pltpu.VMEM((bpw, _DPAD), jnp.float32),
          pltpu.SemaphoreType.DMA,
      ],
  )
  def embed(labels_hbm, dropw_hbm, table_hbm, out_hbm,
            dropw_v, idx_v, null_v, rows_v, sem):
    wid = lax.axis_index("s") * _NUM_CORES + lax.axis_index("c")
    base = pl.multiple_of(wid * bpw, bpw)
    base4 = pl.multiple_of(wid * (bpw // 4), bpw // 4)
    for j in range(nchunk):
      pltpu.sync_copy(labels_hbm.at[pl.ds(base + j * _IDX_CHUNK, _IDX_CHUNK)],
                      idx_v.at[j])
    pltpu.sync_copy(dropw_hbm.at[pl.ds(base4, bpw // 4)], dropw_v)
    pltpu.sync_copy(table_hbm.at[pl.ds(_NULL_INDEX, 1)], null_v)
    copies = []
    for j in range(nchunk):
      copies.append(
          pltpu.async_copy(
              table_hbm.at[idx_v.at[j]],
              rows_v.at[pl.ds(j * _IDX_CHUNK, _IDX_CHUNK)],
              sem,
          ))
    for c in copies:
      c.wait()

    null_cg = [null_v[0, pl.ds(cg * _LANES, _LANES)] for cg in range(ncg)]

    def blend_group(i, carry):
      for j in range(_LANES):
        word_idx = jnp.full((_LANES,), i * 4 + j // 4, jnp.int32)
        flagw = plsc.load_gather(dropw_v, [word_idx])
        flag = (flagw >> (8 * (j % 4))) & 0xFF
        pred = flag != 0
        r = i * _LANES + j
        for cg in range(ncg):
          cur = rows_v[r, pl.ds(cg * _LANES, _LANES)]
          rows_v[r, pl.ds(cg * _LANES, _LANES)] = jnp.where(
              pred, null_cg[cg], cur)
      return carry

    lax.fori_loop(0, bpw // _LANES, blend_group, 0)
    pltpu.sync_copy(rows_v, out_hbm.at[pl.ds(base, bpw)])

  return embed


def _embed_call(labels, dropw, table_pad):
  B, = labels.shape
  out_pad = _build_embed(B, 64)(labels, dropw, table_pad)
  return out_pad[:, :64]


def kernel(labels, drop_labels, embedding_table):
  labels = labels.astype(jnp.int32)
  drop8 = drop_labels.view(jnp.int8)
  dropw = jax.lax.bitcast_convert_type(
      drop8.reshape(drop8.shape[0] // 4, 4), jnp.int32)
  zpad = jax.lax.optimization_barrier(jnp.float32(0.0))
  table_pad = jax.lax.pad(embedding_table, zpad, ((0, 0, 0), (0, _DPAD - 64, 0)))
  return _embed_call(labels, dropw, table_pad)
